# baseline (speedup 1.0000x reference)
import functools

import jax
import jax.numpy as jnp
from jax import lax
from jax.experimental import pallas as pl
from jax.experimental.pallas import tpu as pltpu
from jax.experimental.pallas import tpu_sc as plsc

OMEGA = 30.0
T = 4096
D = 1024
L = 512
E = 8
O = 1024
K = 2

BT = 256
NB = 39
P_PAD = NB * BT

NC = 2
NS = 16
NW = NC * NS
TPW = T // NW
CH = 128
NCH = TPW // CH


D2 = D // 2
L2 = L // 2
O2 = O // 2


def _pack_bf16(v):
    b = lax.bitcast_convert_type(
        v.astype(jnp.bfloat16).astype(jnp.float32), jnp.int32)
    n = v.shape[1] // 2
    return b[:, :n] | lax.shift_right_logical(b[:, n:], 16)


def _unpack_f32(p):
    hi = lax.bitcast_convert_type(p & jnp.int32(-65536), jnp.float32)
    lo = lax.bitcast_convert_type(lax.shift_left(p, 16), jnp.float32)
    return jnp.concatenate([hi, lo], axis=1)


def _unpack_bf16(p):
    return _unpack_f32(p).astype(jnp.bfloat16)


def _inclusive_scan_rows(a):
    d = 1
    n = a.shape[0]
    while d < n:
        pad = jnp.zeros((d, a.shape[1]), a.dtype)
        a = a + jnp.concatenate([pad, a[:-d]], axis=0)
        d *= 2
    return a


def _exclusive_scan_lanes(c):
    acc = c
    d = 1
    n = c.shape[1]
    while d < n:
        pad = jnp.zeros((1, d), c.dtype)
        acc = acc + jnp.concatenate([pad, acc[:, :-d]], axis=1)
        d *= 2
    return acc - c


def _route_body(x_ref, l_ref, gw_ref, gb_ref, dest_ref, wts_ref, cnt_ref,
                xbf_ref, lbf_ref):
    x = x_ref[...]
    xbf_ref[...] = _pack_bf16(x)
    lbf_ref[...] = _pack_bf16(l_ref[...])
    logits = lax.dot_general(x, gw_ref[...], (((1,), (1,)), ((), ())),
                             preferred_element_type=jnp.float32)
    logits = logits + gb_ref[...]
    iota_e = lax.broadcasted_iota(jnp.int32, (T, E), 1)

    m1 = jnp.max(logits, axis=1, keepdims=True)
    e0 = jnp.min(jnp.where(logits == m1, iota_e, E), axis=1, keepdims=True)
    masked = jnp.where(iota_e == e0, -jnp.inf, logits)
    m2 = jnp.max(masked, axis=1, keepdims=True)
    e1 = jnp.min(jnp.where(masked == m2, iota_e, E), axis=1, keepdims=True)

    w0 = 1.0 / (1.0 + jnp.exp(m2 - m1))
    w1 = 1.0 - w0

    oh0 = (iota_e == e0).astype(jnp.int32)
    oh1 = (iota_e == e1).astype(jnp.int32)
    ohc = oh0 + oh1
    incl = _inclusive_scan_rows(ohc)
    excl = incl - ohc
    counts = incl[T - 1:T, :]

    padded = ((counts + (BT - 1)) // BT) * BT
    padoff = _exclusive_scan_lanes(padded)

    rank0 = jnp.sum(oh0 * excl, axis=1, keepdims=True)
    rank1 = jnp.sum(oh1 * excl, axis=1, keepdims=True)
    base0 = jnp.sum(oh0 * padoff, axis=1, keepdims=True)
    base1 = jnp.sum(oh1 * padoff, axis=1, keepdims=True)

    dest_ref[...] = jnp.concatenate([base0 + rank0, base1 + rank1], axis=1)
    wts_ref[...] = jnp.concatenate([w0, w1], axis=1)
    cnt_ref[...] = counts


def _route(x, latents, gate_W, gate_b2):
    return pl.pallas_call(
        _route_body,
        out_shape=(
            jax.ShapeDtypeStruct((T, K), jnp.int32),
            jax.ShapeDtypeStruct((T, K), jnp.float32),
            jax.ShapeDtypeStruct((1, E), jnp.int32),
            jax.ShapeDtypeStruct((T, D2), jnp.int32),
            jax.ShapeDtypeStruct((T, L2), jnp.int32),
        ),
    )(x, latents, gate_W, gate_b2)


def _sc_mesh():
    return plsc.VectorSubcoreMesh(core_axis_name="c", subcore_axis_name="s",
                                  num_cores=NC, num_subcores=NS)


def _dispatch(x, latents, dest4):
    @functools.partial(
        pl.kernel,
        out_type=(
            jax.ShapeDtypeStruct((P_PAD, D2), jnp.int32),
            jax.ShapeDtypeStruct((P_PAD, L2), jnp.int32),
        ),
        mesh=_sc_mesh(),
        scratch_types=[
            pltpu.VMEM((CH, D2), jnp.int32),
            pltpu.VMEM((CH, L2), jnp.int32),
            pltpu.VMEM((CH,), jnp.int32),
            pltpu.VMEM((CH,), jnp.int32),
            pltpu.SemaphoreType.DMA,
            pltpu.SemaphoreType.DMA,
        ],
    )
    def k(x_hbm, l_hbm, d_hbm, xs_hbm, ls_hbm, xv, lv, iv0, iv1, semL, semS):
        wid = lax.axis_index("s") * NC + lax.axis_index("c")
        base = wid * TPW
        loads = [
            pltpu.async_copy(x_hbm.at[pl.ds(base, CH)], xv, semL),
            pltpu.async_copy(l_hbm.at[pl.ds(base, CH)], lv, semL),
            pltpu.async_copy(d_hbm.at[0].at[wid].at[0], iv0, semL),
            pltpu.async_copy(d_hbm.at[1].at[wid].at[0], iv1, semL),
        ]
        for cp in loads:
            cp.wait()
        stores = [
            pltpu.async_copy(xv, xs_hbm.at[iv0], semS),
            pltpu.async_copy(lv, ls_hbm.at[iv0], semS),
            pltpu.async_copy(xv, xs_hbm.at[iv1], semS),
            pltpu.async_copy(lv, ls_hbm.at[iv1], semS),
        ]
        for cp in stores:
            cp.wait()

    return k(x, latents, dest4)


_INV_PI = 0.3183098861837907
_PI_HI = 3.140625
_PI_LO = 9.676535897932795e-04
_S1 = 0.9999966010501739
_S3 = -0.1666482356167327
_S5 = 0.008306286141814084
_S7 = -0.00018362748576797316


def _fast_sin(u):
    k = lax.round(u * _INV_PI, lax.RoundingMethod.TO_NEAREST_EVEN)
    parity = lax.shift_left(k.astype(jnp.int32) & 1, 31)
    r = (u - k * _PI_HI) - k * _PI_LO
    r2 = r * r
    p = r * (_S1 + r2 * (_S3 + r2 * (_S5 + r2 * _S7)))
    return lax.bitcast_convert_type(
        lax.bitcast_convert_type(p, jnp.int32) ^ parity, jnp.float32)


def _grouped_body(blk_ref, nblk_ref, xs_ref, ls_ref, w_ref, b_ref, wl_ref,
                  bl_ref, eo_ref, wbf_ref, wlbf_ref):
    b = pl.program_id(0)

    @pl.when(b < nblk_ref[0])
    def _():
        first = jnp.logical_or(
            b == 0, blk_ref[b] != blk_ref[jnp.maximum(b - 1, 0)])

        @pl.when(first)
        def _():
            wbf_ref[...] = w_ref[0].astype(jnp.bfloat16)
            wlbf_ref[...] = wl_ref[0].astype(jnp.bfloat16)

        a = lax.dot_general(_unpack_bf16(xs_ref[...]), wbf_ref[...],
                            (((1,), (1,)), ((), ())),
                            preferred_element_type=jnp.float32)
        a = a + b_ref[0]
        t = lax.dot_general(_unpack_bf16(ls_ref[...]), wlbf_ref[...],
                            (((1,), (1,)), ((), ())),
                            preferred_element_type=jnp.float32)
        t = t + bl_ref[0]
        g = t[:, :O]
        h = t[:, O:]
        eo_ref[...] = _pack_bf16(_fast_sin(OMEGA * a * g + h))


def _grouped(blk_e, nblk, xs, ls, W_e, b_e3, Wl_e, bl_e3):
    grid_spec = pltpu.PrefetchScalarGridSpec(
        num_scalar_prefetch=2,
        grid=(NB,),
        in_specs=[
            pl.BlockSpec((BT, D2), lambda b, blk, nb: (b, 0)),
            pl.BlockSpec((BT, L2), lambda b, blk, nb: (b, 0)),
            pl.BlockSpec((1, O, D), lambda b, blk, nb: (blk[b], 0, 0)),
            pl.BlockSpec((1, 1, O), lambda b, blk, nb: (blk[b], 0, 0)),
            pl.BlockSpec((1, 2 * O, L), lambda b, blk, nb: (blk[b], 0, 0)),
            pl.BlockSpec((1, 1, 2 * O), lambda b, blk, nb: (blk[b], 0, 0)),
        ],
        out_specs=pl.BlockSpec((BT, O2), lambda b, blk, nb: (b, 0)),
        scratch_shapes=[
            pltpu.VMEM((O, D), jnp.bfloat16),
            pltpu.VMEM((2 * O, L), jnp.bfloat16),
        ],
    )
    return pl.pallas_call(
        _grouped_body,
        grid_spec=grid_spec,
        out_shape=jax.ShapeDtypeStruct((P_PAD, O2), jnp.int32),
    )(blk_e, nblk, xs, ls, W_e, b_e3, Wl_e, bl_e3)


def _gather(eo, dest4):
    @functools.partial(
        pl.kernel,
        out_type=(
            jax.ShapeDtypeStruct((T, O2), jnp.int32),
            jax.ShapeDtypeStruct((T, O2), jnp.int32),
        ),
        mesh=_sc_mesh(),
        scratch_types=[
            pltpu.VMEM((CH // 2, O2), jnp.int32),
            pltpu.VMEM((CH // 2, O2), jnp.int32),
            pltpu.VMEM((CH,), jnp.int32),
            pltpu.VMEM((CH,), jnp.int32),
            pltpu.SemaphoreType.DMA,
            pltpu.SemaphoreType.DMA,
        ],
    )
    def k(eo_hbm, d_hbm, g0_hbm, g1_hbm, r0, r1, iv0, iv1, semA, semB):
        wid = lax.axis_index("s") * NC + lax.axis_index("c")
        base = wid * TPW
        h = CH // 2
        pltpu.async_copy(d_hbm.at[0].at[wid].at[0], iv0, semA).wait()
        pltpu.async_copy(d_hbm.at[1].at[wid].at[0], iv1, semA).wait()
        units = [(iv0, 0, g0_hbm), (iv0, 1, g0_hbm),
                 (iv1, 0, g1_hbm), (iv1, 1, g1_hbm)]
        bufs = (r0, r1)
        sems = (semA, semB)
        pend = pltpu.async_copy(eo_hbm.at[units[0][0].at[pl.ds(0, h)]],
                                bufs[0], sems[0])
        for u in range(4):
            iv, hh, out_h = units[u]
            pend.wait()
            if u < 3:
                niv, nhh, _ = units[u + 1]
                pend = pltpu.async_copy(
                    eo_hbm.at[niv.at[pl.ds(nhh * h, h)]],
                    bufs[(u + 1) % 2], sems[(u + 1) % 2])
            pltpu.sync_copy(bufs[u % 2],
                            out_h.at[pl.ds(base + hh * h, h)])

    return k(eo, dest4)


def _combine_body(g0_ref, g1_ref, wts_ref, o_ref):
    g0 = _unpack_bf16(g0_ref[...]).astype(jnp.float32)
    g1 = _unpack_bf16(g1_ref[...]).astype(jnp.float32)
    o_ref[...] = wts_ref[:, 0:1] * g0 + wts_ref[:, 1:2] * g1


def _combine(g0, g1, wts):
    btc = 512
    return pl.pallas_call(
        _combine_body,
        grid=(T // btc,),
        in_specs=[
            pl.BlockSpec((btc, O2), lambda i: (i, 0)),
            pl.BlockSpec((btc, O2), lambda i: (i, 0)),
            pl.BlockSpec((btc, K), lambda i: (i, 0)),
        ],
        out_specs=pl.BlockSpec((btc, O), lambda i: (i, 0)),
        out_shape=jax.ShapeDtypeStruct((T, O), jnp.float32),
    )(g0, g1, wts)


def kernel(x, latents, gate_W, gate_b, W_e, b_e, Wl_e, bl_e):
    dest, wts, counts, xbf, lbf = _route(x, latents, gate_W,
                                         gate_b.reshape(1, E))

    padded = ((counts[0] + (BT - 1)) // BT) * BT
    ends = jnp.cumsum(padded)
    starts = jnp.arange(NB, dtype=jnp.int32) * BT
    blk_e = jnp.minimum(
        jnp.sum((starts[:, None] >= ends[None, :]).astype(jnp.int32), axis=1),
        E - 1).astype(jnp.int32)

    dest4 = dest.T.reshape(K, NW, NCH, CH)

    nblk = (ends[E - 1] // BT).reshape(1)
    xs, ls = _dispatch(xbf, lbf, dest4)
    eo = _grouped(blk_e, nblk, xs, ls, W_e, b_e.reshape(E, 1, O), Wl_e,
                  bl_e.reshape(E, 1, 2 * O))
    g0, g1 = _gather(eo, dest4)
    out = _combine(g0, g1, wts)
    return (out, latents)


# speedup vs baseline: 1.0038x; 1.0038x over previous
import functools

import jax
import jax.numpy as jnp
from jax import lax
from jax.experimental import pallas as pl
from jax.experimental.pallas import tpu as pltpu
from jax.experimental.pallas import tpu_sc as plsc

OMEGA = 30.0
T = 4096
D = 1024
L = 512
E = 8
O = 1024
K = 2

BT = 256
NB = 39
P_PAD = NB * BT

NC = 2
NS = 16
NW = NC * NS
TPW = T // NW
CH = 128
NCH = TPW // CH


D2 = D // 2
L2 = L // 2
O2 = O // 2


def _pack_bf16(v):
    b = lax.bitcast_convert_type(
        v.astype(jnp.bfloat16).astype(jnp.float32), jnp.int32)
    n = v.shape[1] // 2
    return b[:, :n] | lax.shift_right_logical(b[:, n:], 16)


def _unpack_f32(p):
    hi = lax.bitcast_convert_type(p & jnp.int32(-65536), jnp.float32)
    lo = lax.bitcast_convert_type(lax.shift_left(p, 16), jnp.float32)
    return jnp.concatenate([hi, lo], axis=1)


def _unpack_bf16(p):
    return _unpack_f32(p).astype(jnp.bfloat16)


def _inclusive_scan_rows(a):
    d = 1
    n = a.shape[0]
    while d < n:
        pad = jnp.zeros((d, a.shape[1]), a.dtype)
        a = a + jnp.concatenate([pad, a[:-d]], axis=0)
        d *= 2
    return a


def _exclusive_scan_lanes(c):
    acc = c
    d = 1
    n = c.shape[1]
    while d < n:
        pad = jnp.zeros((1, d), c.dtype)
        acc = acc + jnp.concatenate([pad, acc[:, :-d]], axis=1)
        d *= 2
    return acc - c


BTK = 512
GT = T // BTK


def _route_body(x_ref, l_ref, gw_ref, gb_ref, dest_ref, wts_ref, cnt_ref,
                xbf_ref, lbf_ref, rank_scr, carry_ref):
    b = pl.program_id(0)

    @pl.when(b == 0)
    def _():
        carry_ref[...] = jnp.zeros((1, E), jnp.int32)

    x = x_ref[...]
    xbf_ref[...] = _pack_bf16(x)
    lbf_ref[...] = _pack_bf16(l_ref[...])
    logits = lax.dot_general(x, gw_ref[...], (((1,), (1,)), ((), ())),
                             preferred_element_type=jnp.float32)
    logits = logits + gb_ref[...]
    iota_e = lax.broadcasted_iota(jnp.int32, (BTK, E), 1)

    m1 = jnp.max(logits, axis=1, keepdims=True)
    e0 = jnp.min(jnp.where(logits == m1, iota_e, E), axis=1, keepdims=True)
    masked = jnp.where(iota_e == e0, -jnp.inf, logits)
    m2 = jnp.max(masked, axis=1, keepdims=True)
    e1 = jnp.min(jnp.where(masked == m2, iota_e, E), axis=1, keepdims=True)

    w0 = 1.0 / (1.0 + jnp.exp(m2 - m1))
    w1 = 1.0 - w0
    wts_ref[...] = jnp.concatenate([w0, w1], axis=1)

    oh0 = (iota_e == e0).astype(jnp.int32)
    oh1 = (iota_e == e1).astype(jnp.int32)
    ohc = oh0 + oh1
    incl = _inclusive_scan_rows(ohc)
    excl = incl - ohc + carry_ref[...]
    carry_ref[...] = carry_ref[...] + incl[BTK - 1:BTK, :]

    rank0 = jnp.sum(oh0 * excl, axis=1, keepdims=True)
    rank1 = jnp.sum(oh1 * excl, axis=1, keepdims=True)
    rank_scr[pl.ds(b * BTK, BTK), :] = jnp.concatenate(
        [rank0 | lax.shift_left(e0, 16), rank1 | lax.shift_left(e1, 16)],
        axis=1)

    @pl.when(b == GT - 1)
    def _():
        counts = carry_ref[...]
        cnt_ref[...] = counts
        padded = ((counts + (BT - 1)) // BT) * BT
        padoff = _exclusive_scan_lanes(padded)
        s = rank_scr[...]
        e_arr = lax.shift_right_logical(s, 16)
        r = s & 0xFFFF
        pd = jnp.zeros((T, K), jnp.int32)
        for j in range(E):
            pd = pd + jnp.where(e_arr == j, padoff[:, j:j + 1], 0)
        dest_ref[...] = r + pd


def _route(x, latents, gate_W, gate_b2):
    return pl.pallas_call(
        _route_body,
        grid=(GT,),
        in_specs=[
            pl.BlockSpec((BTK, D), lambda b: (b, 0)),
            pl.BlockSpec((BTK, L), lambda b: (b, 0)),
            pl.BlockSpec((E, D), lambda b: (0, 0)),
            pl.BlockSpec((1, E), lambda b: (0, 0)),
        ],
        out_specs=(
            pl.BlockSpec((T, K), lambda b: (0, 0)),
            pl.BlockSpec((BTK, K), lambda b: (b, 0)),
            pl.BlockSpec((1, E), lambda b: (0, 0)),
            pl.BlockSpec((BTK, D2), lambda b: (b, 0)),
            pl.BlockSpec((BTK, L2), lambda b: (b, 0)),
        ),
        out_shape=(
            jax.ShapeDtypeStruct((T, K), jnp.int32),
            jax.ShapeDtypeStruct((T, K), jnp.float32),
            jax.ShapeDtypeStruct((1, E), jnp.int32),
            jax.ShapeDtypeStruct((T, D2), jnp.int32),
            jax.ShapeDtypeStruct((T, L2), jnp.int32),
        ),
        scratch_shapes=[
            pltpu.VMEM((T, K), jnp.int32),
            pltpu.VMEM((1, E), jnp.int32),
        ],
    )(x, latents, gate_W, gate_b2)


def _sc_mesh():
    return plsc.VectorSubcoreMesh(core_axis_name="c", subcore_axis_name="s",
                                  num_cores=NC, num_subcores=NS)


def _dispatch(x, latents, dest4):
    @functools.partial(
        pl.kernel,
        out_type=(
            jax.ShapeDtypeStruct((P_PAD, D2), jnp.int32),
            jax.ShapeDtypeStruct((P_PAD, L2), jnp.int32),
        ),
        mesh=_sc_mesh(),
        scratch_types=[
            pltpu.VMEM((CH, D2), jnp.int32),
            pltpu.VMEM((CH, L2), jnp.int32),
            pltpu.VMEM((CH,), jnp.int32),
            pltpu.VMEM((CH,), jnp.int32),
            pltpu.SemaphoreType.DMA,
            pltpu.SemaphoreType.DMA,
        ],
    )
    def k(x_hbm, l_hbm, d_hbm, xs_hbm, ls_hbm, xv, lv, iv0, iv1, semL, semS):
        wid = lax.axis_index("s") * NC + lax.axis_index("c")
        base = wid * TPW
        loads = [
            pltpu.async_copy(x_hbm.at[pl.ds(base, CH)], xv, semL),
            pltpu.async_copy(l_hbm.at[pl.ds(base, CH)], lv, semL),
            pltpu.async_copy(d_hbm.at[0].at[wid].at[0], iv0, semL),
            pltpu.async_copy(d_hbm.at[1].at[wid].at[0], iv1, semL),
        ]
        for cp in loads:
            cp.wait()
        stores = [
            pltpu.async_copy(xv, xs_hbm.at[iv0], semS),
            pltpu.async_copy(lv, ls_hbm.at[iv0], semS),
            pltpu.async_copy(xv, xs_hbm.at[iv1], semS),
            pltpu.async_copy(lv, ls_hbm.at[iv1], semS),
        ]
        for cp in stores:
            cp.wait()

    return k(x, latents, dest4)


_INV_PI = 0.3183098861837907
_PI_HI = 3.140625
_PI_LO = 9.676535897932795e-04
_S1 = 0.9999966010501739
_S3 = -0.1666482356167327
_S5 = 0.008306286141814084
_S7 = -0.00018362748576797316


def _fast_sin(u):
    k = lax.round(u * _INV_PI, lax.RoundingMethod.TO_NEAREST_EVEN)
    parity = lax.shift_left(k.astype(jnp.int32) & 1, 31)
    r = (u - k * _PI_HI) - k * _PI_LO
    r2 = r * r
    p = r * (_S1 + r2 * (_S3 + r2 * (_S5 + r2 * _S7)))
    return lax.bitcast_convert_type(
        lax.bitcast_convert_type(p, jnp.int32) ^ parity, jnp.float32)


def _grouped_body(blk_ref, nblk_ref, xs_ref, ls_ref, w_ref, b_ref, wl_ref,
                  bl_ref, eo_ref, wbf_ref, wlbf_ref):
    b = pl.program_id(0)

    @pl.when(b < nblk_ref[0])
    def _():
        first = jnp.logical_or(
            b == 0, blk_ref[b] != blk_ref[jnp.maximum(b - 1, 0)])

        @pl.when(first)
        def _():
            wbf_ref[...] = w_ref[0].astype(jnp.bfloat16)
            wlbf_ref[...] = wl_ref[0].astype(jnp.bfloat16)

        a = lax.dot_general(_unpack_bf16(xs_ref[...]), wbf_ref[...],
                            (((1,), (1,)), ((), ())),
                            preferred_element_type=jnp.float32)
        a = a + b_ref[0]
        t = lax.dot_general(_unpack_bf16(ls_ref[...]), wlbf_ref[...],
                            (((1,), (1,)), ((), ())),
                            preferred_element_type=jnp.float32)
        t = t + bl_ref[0]
        g = t[:, :O]
        h = t[:, O:]
        eo_ref[...] = _pack_bf16(_fast_sin(OMEGA * a * g + h))


def _grouped(blk_e, nblk, xs, ls, W_e, b_e3, Wl_e, bl_e3):
    grid_spec = pltpu.PrefetchScalarGridSpec(
        num_scalar_prefetch=2,
        grid=(NB,),
        in_specs=[
            pl.BlockSpec((BT, D2), lambda b, blk, nb: (b, 0)),
            pl.BlockSpec((BT, L2), lambda b, blk, nb: (b, 0)),
            pl.BlockSpec((1, O, D), lambda b, blk, nb: (blk[b], 0, 0)),
            pl.BlockSpec((1, 1, O), lambda b, blk, nb: (blk[b], 0, 0)),
            pl.BlockSpec((1, 2 * O, L), lambda b, blk, nb: (blk[b], 0, 0)),
            pl.BlockSpec((1, 1, 2 * O), lambda b, blk, nb: (blk[b], 0, 0)),
        ],
        out_specs=pl.BlockSpec((BT, O2), lambda b, blk, nb: (b, 0)),
        scratch_shapes=[
            pltpu.VMEM((O, D), jnp.bfloat16),
            pltpu.VMEM((2 * O, L), jnp.bfloat16),
        ],
    )
    return pl.pallas_call(
        _grouped_body,
        grid_spec=grid_spec,
        out_shape=jax.ShapeDtypeStruct((P_PAD, O2), jnp.int32),
    )(blk_e, nblk, xs, ls, W_e, b_e3, Wl_e, bl_e3)


def _gather(eo, dest4):
    @functools.partial(
        pl.kernel,
        out_type=(
            jax.ShapeDtypeStruct((T, O2), jnp.int32),
            jax.ShapeDtypeStruct((T, O2), jnp.int32),
        ),
        mesh=_sc_mesh(),
        scratch_types=[
            pltpu.VMEM((CH // 2, O2), jnp.int32),
            pltpu.VMEM((CH // 2, O2), jnp.int32),
            pltpu.VMEM((CH,), jnp.int32),
            pltpu.VMEM((CH,), jnp.int32),
            pltpu.SemaphoreType.DMA,
            pltpu.SemaphoreType.DMA,
        ],
    )
    def k(eo_hbm, d_hbm, g0_hbm, g1_hbm, r0, r1, iv0, iv1, semA, semB):
        wid = lax.axis_index("s") * NC + lax.axis_index("c")
        base = wid * TPW
        h = CH // 2
        pltpu.async_copy(d_hbm.at[0].at[wid].at[0], iv0, semA).wait()
        pltpu.async_copy(d_hbm.at[1].at[wid].at[0], iv1, semA).wait()
        units = [(iv0, 0, g0_hbm), (iv0, 1, g0_hbm),
                 (iv1, 0, g1_hbm), (iv1, 1, g1_hbm)]
        bufs = (r0, r1)
        sems = (semA, semB)
        pend = pltpu.async_copy(eo_hbm.at[units[0][0].at[pl.ds(0, h)]],
                                bufs[0], sems[0])
        for u in range(4):
            iv, hh, out_h = units[u]
            pend.wait()
            if u < 3:
                niv, nhh, _ = units[u + 1]
                pend = pltpu.async_copy(
                    eo_hbm.at[niv.at[pl.ds(nhh * h, h)]],
                    bufs[(u + 1) % 2], sems[(u + 1) % 2])
            pltpu.sync_copy(bufs[u % 2],
                            out_h.at[pl.ds(base + hh * h, h)])

    return k(eo, dest4)


def _combine_body(g0_ref, g1_ref, wts_ref, o_ref):
    g0 = _unpack_bf16(g0_ref[...]).astype(jnp.float32)
    g1 = _unpack_bf16(g1_ref[...]).astype(jnp.float32)
    o_ref[...] = wts_ref[:, 0:1] * g0 + wts_ref[:, 1:2] * g1


def _combine(g0, g1, wts):
    btc = 512
    return pl.pallas_call(
        _combine_body,
        grid=(T // btc,),
        in_specs=[
            pl.BlockSpec((btc, O2), lambda i: (i, 0)),
            pl.BlockSpec((btc, O2), lambda i: (i, 0)),
            pl.BlockSpec((btc, K), lambda i: (i, 0)),
        ],
        out_specs=pl.BlockSpec((btc, O), lambda i: (i, 0)),
        out_shape=jax.ShapeDtypeStruct((T, O), jnp.float32),
    )(g0, g1, wts)


def kernel(x, latents, gate_W, gate_b, W_e, b_e, Wl_e, bl_e):
    dest, wts, counts, xbf, lbf = _route(x, latents, gate_W,
                                         gate_b.reshape(1, E))

    padded = ((counts[0] + (BT - 1)) // BT) * BT
    ends = jnp.cumsum(padded)
    starts = jnp.arange(NB, dtype=jnp.int32) * BT
    blk_e = jnp.minimum(
        jnp.sum((starts[:, None] >= ends[None, :]).astype(jnp.int32), axis=1),
        E - 1).astype(jnp.int32)

    dest4 = dest.T.reshape(K, NW, NCH, CH)

    nblk = (ends[E - 1] // BT).reshape(1)
    xs, ls = _dispatch(xbf, lbf, dest4)
    eo = _grouped(blk_e, nblk, xs, ls, W_e, b_e.reshape(E, 1, O), Wl_e,
                  bl_e.reshape(E, 1, 2 * O))
    g0, g1 = _gather(eo, dest4)
    out = _combine(g0, g1, wts)
    return (out, latents)
